# baseline (speedup 1.0000x reference)
import jax
import jax.numpy as jnp
from jax.experimental import pallas as pl

_N_Q = 8
_K = 1024
_D = 128
_TILE = 2400


_HALF = _TILE // 2


def _rvq_body(x_ref, cb_ref, cbp_ref, out_ref):
    iota = jax.lax.broadcasted_iota(jnp.int32, (_HALF, _K), 1)

    def dist(r, q):
        return jax.lax.dot_general(
            r, cb_ref[q], (((1,), (1,)), ((), ())),
            preferred_element_type=jnp.float32,
            precision=jax.lax.Precision.DEFAULT)

    def argmin_onehot(r, s2, q):
        cbn = jnp.sum(cb_ref[q] * cb_ref[q], axis=1) * 0.25
        rss = jnp.sum(r * r, axis=1, keepdims=True)
        d = (rss - s2) + cbn[None, :]
        m = jnp.min(d, axis=1, keepdims=True)
        t = jnp.where(d == m, iota, _K)
        idx = jnp.min(t, axis=1, keepdims=True)
        return idx, (iota == idx).astype(jnp.bfloat16)

    def gather_sub(r, onehot, q):
        g = jax.lax.dot_general(
            onehot, cbp_ref[q], (((1,), (0,)), ((), ())),
            preferred_element_type=jnp.float32,
            precision=jax.lax.Precision.DEFAULT)
        return r - ((g[:, :_D] + g[:, _D:2 * _D]) + g[:, 2 * _D:])

    ra = x_ref[:_HALF, :]
    rb = x_ref[_HALF:, :]
    for q in range(_N_Q):
        sa = dist(ra, q)
        sb = dist(rb, q)
        ia, oha = argmin_onehot(ra, sa, q)
        ib, ohb = argmin_onehot(rb, sb, q)
        ra = gather_sub(ra, oha, q)
        rb = gather_sub(rb, ohb, q)
        out_ref[:_HALF, q:q + 1] = ia
        out_ref[_HALF:, q:q + 1] = ib


def kernel(wav_features, codebooks):
    B, T, D = wav_features.shape
    n = B * T
    x = wav_features.reshape(n, D)
    bits = jax.lax.bitcast_convert_type(codebooks, jnp.uint32)
    hi = jax.lax.bitcast_convert_type(bits & jnp.uint32(0xFFFF0000),
                                      jnp.float32)
    r1 = codebooks - hi
    r1b = jax.lax.bitcast_convert_type(r1, jnp.uint32)
    mid = jax.lax.bitcast_convert_type(r1b & jnp.uint32(0xFFFF0000),
                                       jnp.float32)
    lo = r1 - mid
    cbp = jnp.concatenate(
        [hi.astype(jnp.bfloat16), mid.astype(jnp.bfloat16),
         lo.astype(jnp.bfloat16)], axis=-1)
    cb2 = codebooks * 2.0
    out = pl.pallas_call(
        _rvq_body,
        grid=(n // _TILE,),
        in_specs=[
            pl.BlockSpec((_TILE, _D), lambda i: (i, 0)),
            pl.BlockSpec((_N_Q, _K, _D), lambda i: (0, 0, 0)),
            pl.BlockSpec((_N_Q, _K, 3 * _D), lambda i: (0, 0, 0)),
        ],
        out_specs=pl.BlockSpec((_TILE, _N_Q), lambda i: (i, 0)),
        out_shape=jax.ShapeDtypeStruct((n, _N_Q), jnp.int32),
    )(x, cb2, cbp)
    return out.T.reshape(_N_Q, B, T)


# speedup vs baseline: 1.1787x; 1.1787x over previous
import jax
import jax.numpy as jnp
from jax.experimental import pallas as pl

_N_Q = 8
_K = 1024
_D = 128
_TILE = 800


_HALF = _TILE // 2


def _rvq_body(x_ref, cb_ref, cbp_ref, out_ref):
    iota = jax.lax.broadcasted_iota(jnp.int32, (_HALF, _K), 1)

    def dist(r, q):
        return jax.lax.dot_general(
            r, cb_ref[q], (((1,), (1,)), ((), ())),
            preferred_element_type=jnp.float32,
            precision=jax.lax.Precision.DEFAULT)

    def argmin_onehot(r, s2, q):
        cbn = jnp.sum(cb_ref[q] * cb_ref[q], axis=1) * 0.25
        rss = jnp.sum(r * r, axis=1, keepdims=True)
        d = (rss - s2) + cbn[None, :]
        m = jnp.min(d, axis=1, keepdims=True)
        t = jnp.where(d == m, iota, _K)
        idx = jnp.min(t, axis=1, keepdims=True)
        return idx, (iota == idx).astype(jnp.bfloat16)

    def gather_sub(r, onehot, q):
        g = jax.lax.dot_general(
            onehot, cbp_ref[q], (((1,), (0,)), ((), ())),
            preferred_element_type=jnp.float32,
            precision=jax.lax.Precision.DEFAULT)
        return r - ((g[:, :_D] + g[:, _D:2 * _D]) + g[:, 2 * _D:])

    ra = x_ref[:_HALF, :]
    rb = x_ref[_HALF:, :]
    for q in range(_N_Q):
        sa = dist(ra, q)
        sb = dist(rb, q)
        ia, oha = argmin_onehot(ra, sa, q)
        ib, ohb = argmin_onehot(rb, sb, q)
        ra = gather_sub(ra, oha, q)
        rb = gather_sub(rb, ohb, q)
        out_ref[:_HALF, q:q + 1] = ia
        out_ref[_HALF:, q:q + 1] = ib


def kernel(wav_features, codebooks):
    B, T, D = wav_features.shape
    n = B * T
    x = wav_features.reshape(n, D)
    bits = jax.lax.bitcast_convert_type(codebooks, jnp.uint32)
    hi = jax.lax.bitcast_convert_type(bits & jnp.uint32(0xFFFF0000),
                                      jnp.float32)
    r1 = codebooks - hi
    r1b = jax.lax.bitcast_convert_type(r1, jnp.uint32)
    mid = jax.lax.bitcast_convert_type(r1b & jnp.uint32(0xFFFF0000),
                                       jnp.float32)
    lo = r1 - mid
    cbp = jnp.concatenate(
        [hi.astype(jnp.bfloat16), mid.astype(jnp.bfloat16),
         lo.astype(jnp.bfloat16)], axis=-1)
    cb2 = codebooks * 2.0
    out = pl.pallas_call(
        _rvq_body,
        grid=(n // _TILE,),
        in_specs=[
            pl.BlockSpec((_TILE, _D), lambda i: (i, 0)),
            pl.BlockSpec((_N_Q, _K, _D), lambda i: (0, 0, 0)),
            pl.BlockSpec((_N_Q, _K, 3 * _D), lambda i: (0, 0, 0)),
        ],
        out_specs=pl.BlockSpec((_TILE, _N_Q), lambda i: (i, 0)),
        out_shape=jax.ShapeDtypeStruct((n, _N_Q), jnp.int32),
    )(x, cb2, cbp)
    return out.T.reshape(_N_Q, B, T)


# speedup vs baseline: 1.2472x; 1.0581x over previous
import jax
import jax.numpy as jnp
from jax.experimental import pallas as pl

_N_Q = 8
_K = 1024
_D = 128
_TILE = 1200


_HALF = _TILE // 2


def _rvq_body(x_ref, cb_ref, cbp_ref, out_ref):
    iota = jax.lax.broadcasted_iota(jnp.int32, (_HALF, _K), 1)

    def dist(r, q):
        return jax.lax.dot_general(
            r, cb_ref[q], (((1,), (1,)), ((), ())),
            preferred_element_type=jnp.float32,
            precision=jax.lax.Precision.DEFAULT)

    def argmin_onehot(r, s2, q):
        cbn = jnp.sum(cb_ref[q] * cb_ref[q], axis=1) * 0.25
        rss = jnp.sum(r * r, axis=1, keepdims=True)
        d = (rss - s2) + cbn[None, :]
        m = jnp.min(d, axis=1, keepdims=True)
        t = jnp.where(d == m, iota, _K)
        idx = jnp.min(t, axis=1, keepdims=True)
        return idx, (iota == idx).astype(jnp.bfloat16)

    def gather_sub(r, onehot, q):
        g = jax.lax.dot_general(
            onehot, cbp_ref[q], (((1,), (0,)), ((), ())),
            preferred_element_type=jnp.float32,
            precision=jax.lax.Precision.DEFAULT)
        return r - ((g[:, :_D] + g[:, _D:2 * _D]) + g[:, 2 * _D:])

    ra = x_ref[:_HALF, :]
    rb = x_ref[_HALF:, :]
    for q in range(_N_Q):
        sa = dist(ra, q)
        sb = dist(rb, q)
        ia, oha = argmin_onehot(ra, sa, q)
        ib, ohb = argmin_onehot(rb, sb, q)
        ra = gather_sub(ra, oha, q)
        rb = gather_sub(rb, ohb, q)
        out_ref[:_HALF, q:q + 1] = ia
        out_ref[_HALF:, q:q + 1] = ib


def kernel(wav_features, codebooks):
    B, T, D = wav_features.shape
    n = B * T
    x = wav_features.reshape(n, D)
    bits = jax.lax.bitcast_convert_type(codebooks, jnp.uint32)
    hi = jax.lax.bitcast_convert_type(bits & jnp.uint32(0xFFFF0000),
                                      jnp.float32)
    r1 = codebooks - hi
    r1b = jax.lax.bitcast_convert_type(r1, jnp.uint32)
    mid = jax.lax.bitcast_convert_type(r1b & jnp.uint32(0xFFFF0000),
                                       jnp.float32)
    lo = r1 - mid
    cbp = jnp.concatenate(
        [hi.astype(jnp.bfloat16), mid.astype(jnp.bfloat16),
         lo.astype(jnp.bfloat16)], axis=-1)
    cb2 = codebooks * 2.0
    out = pl.pallas_call(
        _rvq_body,
        grid=(n // _TILE,),
        in_specs=[
            pl.BlockSpec((_TILE, _D), lambda i: (i, 0)),
            pl.BlockSpec((_N_Q, _K, _D), lambda i: (0, 0, 0)),
            pl.BlockSpec((_N_Q, _K, 3 * _D), lambda i: (0, 0, 0)),
        ],
        out_specs=pl.BlockSpec((_TILE, _N_Q), lambda i: (i, 0)),
        out_shape=jax.ShapeDtypeStruct((n, _N_Q), jnp.int32),
    )(x, cb2, cbp)
    return out.T.reshape(_N_Q, B, T)
